# baseline (speedup 1.0000x reference)
import jax
import jax.numpy as jnp
from jax.experimental import pallas as pl
from jax.experimental.pallas import tpu as pltpu

_N = 11
_NO = 3


def _spec_kernel(t_hbm, fl_hbm, f_hbm, e_hbm, x_hbm, o_hbm,
                 tvm, flvm, fvm, evm, xbuf, obuf,
                 m_sem, h1_sem, in_sems, out_sems):
    B, L, D = x_hbm.shape
    H = L // 2

    def in_copy(i, s):
        return pltpu.make_async_copy(x_hbm.at[i], xbuf.at[s], in_sems.at[s])

    def out_half(i, so, h):
        sl = pl.ds(h * H, H)
        return pltpu.make_async_copy(obuf.at[so, sl], o_hbm.at[i, sl],
                                     out_sems.at[so])

    mask_copies = [
        pltpu.make_async_copy(t_hbm, tvm, m_sem),
        pltpu.make_async_copy(fl_hbm, flvm, m_sem),
        pltpu.make_async_copy(f_hbm, fvm, m_sem),
        pltpu.make_async_copy(e_hbm, evm, m_sem),
    ]
    for c in mask_copies:
        c.start()
    def in_half(s, h, sem):
        sl = pl.ds(h * H, H)
        return pltpu.make_async_copy(x_hbm.at[s, sl], xbuf.at[s, sl], sem)

    c0h0 = in_half(0, 0, in_sems.at[0])
    c0h1 = in_half(0, 1, h1_sem)
    c0h0.start()
    c0h1.start()
    for s in range(1, _N):
        in_copy(s, s).start()

    eye = obuf.at[1]
    rows = jax.lax.broadcasted_iota(jnp.int32, (L, L), 0)
    cols = jax.lax.broadcasted_iota(jnp.int32, (L, L), 1)
    eye[...] = jnp.where(rows == cols, jnp.float32(1), jnp.float32(0))

    for c in mask_copies:
        c.wait()
    e = evm[...]

    tf = (tvm[...].astype(jnp.float32) *
          flvm[...].astype(jnp.float32))
    tmat = jax.lax.dot_general(
        eye[...], tf, (((1,), (1,)), ((), ())),
        preferred_element_type=jnp.float32)
    fm = fvm[...].astype(jnp.float32)

    def sample_masks(i):
        ohl = (jax.lax.broadcasted_iota(jnp.int32, (1, B), 1) == i
               ).astype(jnp.float32)
        t = jnp.sum(tmat * ohl, axis=1, keepdims=True) != 0.0
        ohs = (jax.lax.broadcasted_iota(jnp.int32, (B, 1), 0) == i
               ).astype(jnp.float32)
        f = jnp.sum(fm * ohs, axis=0, keepdims=True) != 0.0
        return t, f

    t0, f0 = sample_masks(0)
    c0h0.wait()
    obuf[0, : H] = jnp.where(f0, jnp.float32(0.0),
                             jnp.where(t0[:H], e, xbuf[0, : H]))
    out_half(0, 0, 0).start()
    c0h1.wait()
    obuf[0, H:] = jnp.where(f0, jnp.float32(0.0),
                            jnp.where(t0[H:], e, xbuf[0, H:]))
    out_half(0, 0, 1).start()
    in_copy(_N, 0).start()

    def step(j, carry):
        for k in range(2):
            i = 2 * j + k + 1
            s = jax.lax.rem(i, _N)
            so = jax.lax.rem(i, _NO)

            t, f = sample_masks(i)

            in_copy(i, s).wait()

            @pl.when(i >= _NO)
            def _():
                out_half(i - _NO, so, 0).wait()
                out_half(i - _NO, so, 1).wait()

            obuf[so] = jnp.where(f, jnp.float32(0.0), jnp.where(t, e, xbuf[s]))
            out_half(i, so, 0).start()
            out_half(i, so, 1).start()

            @pl.when(i + _N < B)
            def _():
                in_copy(i + _N, s).start()

        return carry

    jax.lax.fori_loop(0, (B - 1) // 2, step, 0)
    i = B - 1
    s = jax.lax.rem(i, _N)
    so = jax.lax.rem(i, _NO)
    t, f = sample_masks(i)
    in_copy(i, s).wait()
    out_half(i - _NO, so, 0).wait()
    out_half(i - _NO, so, 1).wait()
    obuf[so] = jnp.where(f, jnp.float32(0.0), jnp.where(t, e, xbuf[s]))
    out_half(i, so, 0).start()
    out_half(i, so, 1).start()
    for k in range(_NO):
        i = B - _NO + k
        so = jax.lax.rem(i, _NO)
        out_half(i, so, 0).wait()
        out_half(i, so, 1).wait()


def kernel(x, masked_spec_embed, mask_time, flip_mask, mask_feature):
    B, L, D = x.shape
    e = masked_spec_embed.reshape(1, D).astype(x.dtype)

    f = pl.pallas_call(
        _spec_kernel,
        in_specs=[
            pl.BlockSpec(memory_space=pl.ANY),
            pl.BlockSpec(memory_space=pl.ANY),
            pl.BlockSpec(memory_space=pl.ANY),
            pl.BlockSpec(memory_space=pl.ANY),
            pl.BlockSpec(memory_space=pl.ANY),
        ],
        out_specs=pl.BlockSpec(memory_space=pl.ANY),
        out_shape=jax.ShapeDtypeStruct((B, L, D), x.dtype),
        compiler_params=pltpu.CompilerParams(
            vmem_limit_bytes=64 * 1024 * 1024),
        scratch_shapes=[
            pltpu.VMEM((B, L), jnp.int8),
            pltpu.VMEM((B, L), jnp.int8),
            pltpu.VMEM((B, D), jnp.int8),
            pltpu.VMEM((1, D), x.dtype),
            pltpu.VMEM((_N, L, D), x.dtype),
            pltpu.VMEM((_NO, L, D), x.dtype),
            pltpu.SemaphoreType.DMA,
            pltpu.SemaphoreType.DMA,
            pltpu.SemaphoreType.DMA((_N,)),
            pltpu.SemaphoreType.DMA((_NO,)),
        ],
    )
    bc = lambda m: m.view(jnp.int8)
    return f(bc(mask_time), bc(flip_mask), bc(mask_feature), e, x)
